# baseline (speedup 1.0000x reference)
import jax
import jax.numpy as jnp
from jax import lax
from jax.experimental import pallas as pl
from jax.experimental.pallas import tpu as pltpu

EPS_ = 1.0


def _dice_kernel(x_ref, t_ref, out_ref, acc_ref):
    C, HT, W = x_ref.shape[1:]
    t = t_ref[0]

    def row_group(g, i8):
        tg = t_ref[0, pl.ds(g * 8, 8), :]
        accg = jnp.where(tg == 0, x_ref[0, 0, pl.ds(g * 8, 8), :], 0.0)
        for c in range(1, C):
            accg = accg + jnp.where(tg == c,
                                    x_ref[0, c, pl.ds(g * 8, 8), :], 0.0)
        return i8 + accg

    i8 = lax.fori_loop(0, HT // 8, row_group, jnp.zeros((8, W), jnp.float32))
    i_vec = jnp.sum(i8, axis=0, keepdims=True)

    x2 = x_ref[0].reshape(C * HT, W)
    ones = jnp.ones((1, C * HT), dtype=jnp.float32)
    s_vec = jax.lax.dot_general(
        ones, x2, (((1,), (0,)), ((), ())),
        preferred_element_type=jnp.float32)

    nv_vec = jnp.sum(jnp.where((t >= 0) & (t < C), 1.0, 0.0),
                     axis=0, keepdims=True)

    upd = jnp.concatenate([s_vec, i_vec, nv_vec], axis=0)

    @pl.when(pl.program_id(0) == 0)
    def _init():
        acc_ref[...] = upd

    @pl.when(pl.program_id(0) != 0)
    def _acc():
        acc_ref[...] += upd

    @pl.when(pl.program_id(0) == pl.num_programs(0) - 1)
    def _fin():
        acc = acc_ref[...]
        s = jnp.sum(acc[0])
        inter = jnp.sum(acc[1])
        nvalid = jnp.sum(acc[2])
        union = s + nvalid
        dice = 1.0 - (2.0 * inter + EPS_) / (union + EPS_)
        out_ref[0, 0] = dice / C


def kernel(y_pred, y_true):
    B, C, H, W = y_pred.shape
    HT = 128
    GH = H // HT
    n = B * GH
    out = pl.pallas_call(
        _dice_kernel,
        grid=(n,),
        in_specs=[
            pl.BlockSpec((1, C, HT, W), lambda i: (i // GH, 0, i % GH, 0)),
            pl.BlockSpec((1, HT, W), lambda i: (i // GH, i % GH, 0)),
        ],
        out_specs=pl.BlockSpec((1, 1), lambda i: (0, 0), memory_space=pltpu.SMEM),
        out_shape=jax.ShapeDtypeStruct((1, 1), jnp.float32),
        scratch_shapes=[pltpu.VMEM((3, W), jnp.float32)],
        compiler_params=pltpu.CompilerParams(
            dimension_semantics=("arbitrary",),
        ),
    )(y_pred, y_true.astype(jnp.int32))
    return out[0, 0]


# speedup vs baseline: 1.0775x; 1.0775x over previous
import jax
import jax.numpy as jnp
from jax import lax
from jax.experimental import pallas as pl
from jax.experimental.pallas import tpu as pltpu

EPS_ = 1.0


def _dice_kernel(x_ref, t_ref, out_ref, acc_ref):
    C, HT, W = x_ref.shape[1:]
    t = t_ref[0]

    def row_group(g, i8):
        tg = t_ref[0, pl.ds(g * 8, 8), :]
        accg = jnp.where(tg == 0, x_ref[0, 0, pl.ds(g * 8, 8), :], 0.0)
        for c in range(1, C):
            accg = accg + jnp.where(tg == c,
                                    x_ref[0, c, pl.ds(g * 8, 8), :], 0.0)
        return i8 + accg

    i8 = lax.fori_loop(0, HT // 8, row_group, jnp.zeros((8, W), jnp.float32))
    i_vec = jnp.sum(i8, axis=0, keepdims=True)

    x2 = x_ref[0].reshape(C * HT, W)
    ones = jnp.ones((1, C * HT), dtype=jnp.float32)
    s_vec = jax.lax.dot_general(
        ones, x2, (((1,), (0,)), ((), ())),
        preferred_element_type=jnp.float32)

    nv_vec = jnp.sum(jnp.where((t >= 0) & (t < C), 1.0, 0.0),
                     axis=0, keepdims=True)

    upd = jnp.concatenate([s_vec, i_vec, nv_vec], axis=0)

    @pl.when(pl.program_id(0) == 0)
    def _init():
        acc_ref[...] = upd

    @pl.when(pl.program_id(0) != 0)
    def _acc():
        acc_ref[...] += upd

    @pl.when(pl.program_id(0) == pl.num_programs(0) - 1)
    def _fin():
        acc = acc_ref[...]
        s = jnp.sum(acc[0])
        inter = jnp.sum(acc[1])
        nvalid = jnp.sum(acc[2])
        union = s + nvalid
        dice = 1.0 - (2.0 * inter + EPS_) / (union + EPS_)
        out_ref[0, 0] = dice / C


def kernel(y_pred, y_true):
    B, C, H, W = y_pred.shape
    HT = 512
    GH = H // HT
    n = B * GH
    out = pl.pallas_call(
        _dice_kernel,
        grid=(n,),
        in_specs=[
            pl.BlockSpec((1, C, HT, W), lambda i: (i // GH, 0, i % GH, 0)),
            pl.BlockSpec((1, HT, W), lambda i: (i // GH, i % GH, 0)),
        ],
        out_specs=pl.BlockSpec((1, 1), lambda i: (0, 0), memory_space=pltpu.SMEM),
        out_shape=jax.ShapeDtypeStruct((1, 1), jnp.float32),
        scratch_shapes=[pltpu.VMEM((3, W), jnp.float32)],
        compiler_params=pltpu.CompilerParams(
            dimension_semantics=("arbitrary",),
        ),
    )(y_pred, y_true.astype(jnp.int32))
    return out[0, 0]


# speedup vs baseline: 1.1559x; 1.0727x over previous
import jax
import jax.numpy as jnp
from jax import lax
from jax.experimental import pallas as pl
from jax.experimental.pallas import tpu as pltpu

EPS_ = 1.0


def _dice_kernel(x_ref, t_ref, out_ref, acc_ref):
    C, HT, W = x_ref.shape[1:]

    def row_group(g, carry):
        i8, s8, nv8 = carry
        tg = t_ref[0, pl.ds(g * 8, 8), :]
        xg = x_ref[0, 0, pl.ds(g * 8, 8), :]
        accg = jnp.where(tg == 0, xg, 0.0)
        sg = xg
        for c in range(1, C):
            xg = x_ref[0, c, pl.ds(g * 8, 8), :]
            accg = accg + jnp.where(tg == c, xg, 0.0)
            sg = sg + xg
        nv8 = nv8 + jnp.where((tg >= 0) & (tg < C), 1.0, 0.0)
        return (i8 + accg, s8 + sg, nv8)

    z8 = jnp.zeros((8, W), jnp.float32)
    i8, s8, nv8 = lax.fori_loop(0, HT // 8, row_group, (z8, z8, z8))
    i_vec = jnp.sum(i8, axis=0, keepdims=True)
    s_vec = jnp.sum(s8, axis=0, keepdims=True)
    nv_vec = jnp.sum(nv8, axis=0, keepdims=True)

    upd = jnp.concatenate([s_vec, i_vec, nv_vec], axis=0)

    @pl.when(pl.program_id(0) == 0)
    def _init():
        acc_ref[...] = upd

    @pl.when(pl.program_id(0) != 0)
    def _acc():
        acc_ref[...] += upd

    @pl.when(pl.program_id(0) == pl.num_programs(0) - 1)
    def _fin():
        acc = acc_ref[...]
        s = jnp.sum(acc[0])
        inter = jnp.sum(acc[1])
        nvalid = jnp.sum(acc[2])
        union = s + nvalid
        dice = 1.0 - (2.0 * inter + EPS_) / (union + EPS_)
        out_ref[0, 0] = dice / C


def kernel(y_pred, y_true):
    B, C, H, W = y_pred.shape
    HT = 256
    GH = H // HT
    n = B * GH
    out = pl.pallas_call(
        _dice_kernel,
        grid=(n,),
        in_specs=[
            pl.BlockSpec((1, C, HT, W), lambda i: (i // GH, 0, i % GH, 0)),
            pl.BlockSpec((1, HT, W), lambda i: (i // GH, i % GH, 0)),
        ],
        out_specs=pl.BlockSpec((1, 1), lambda i: (0, 0), memory_space=pltpu.SMEM),
        out_shape=jax.ShapeDtypeStruct((1, 1), jnp.float32),
        scratch_shapes=[pltpu.VMEM((3, W), jnp.float32)],
        compiler_params=pltpu.CompilerParams(
            dimension_semantics=("arbitrary",),
        ),
    )(y_pred, y_true.astype(jnp.int32))
    return out[0, 0]


# speedup vs baseline: 1.1684x; 1.0109x over previous
import jax
import jax.numpy as jnp
from jax import lax
from jax.experimental import pallas as pl
from jax.experimental.pallas import tpu as pltpu

EPS_ = 1.0


def _dice_kernel(x_ref, t_ref, out_ref, acc_ref):
    C, HT, W = x_ref.shape[1:]

    def row_group(g, carry):
        i8, s8, nv8 = carry
        tg = t_ref[0, pl.ds(g * 16, 16), :]
        xg = x_ref[0, 0, pl.ds(g * 16, 16), :]
        accg = jnp.where(tg == 0, xg, 0.0)
        sg = xg
        for c in range(1, C):
            xg = x_ref[0, c, pl.ds(g * 16, 16), :]
            accg = accg + jnp.where(tg == c, xg, 0.0)
            sg = sg + xg
        nv8 = nv8 + jnp.where((tg >= 0) & (tg < C), 1.0, 0.0)
        return (i8 + accg, s8 + sg, nv8)

    z8 = jnp.zeros((16, W), jnp.float32)
    i8, s8, nv8 = lax.fori_loop(0, HT // 16, row_group, (z8, z8, z8))
    i_vec = jnp.sum(i8, axis=0, keepdims=True)
    s_vec = jnp.sum(s8, axis=0, keepdims=True)
    nv_vec = jnp.sum(nv8, axis=0, keepdims=True)

    upd = jnp.concatenate([s_vec, i_vec, nv_vec], axis=0)

    @pl.when(pl.program_id(0) == 0)
    def _init():
        acc_ref[...] = upd

    @pl.when(pl.program_id(0) != 0)
    def _acc():
        acc_ref[...] += upd

    @pl.when(pl.program_id(0) == pl.num_programs(0) - 1)
    def _fin():
        acc = acc_ref[...]
        s = jnp.sum(acc[0])
        inter = jnp.sum(acc[1])
        nvalid = jnp.sum(acc[2])
        union = s + nvalid
        dice = 1.0 - (2.0 * inter + EPS_) / (union + EPS_)
        out_ref[0, 0] = dice / C


def kernel(y_pred, y_true):
    B, C, H, W = y_pred.shape
    HT = 256
    GH = H // HT
    n = B * GH
    out = pl.pallas_call(
        _dice_kernel,
        grid=(n,),
        in_specs=[
            pl.BlockSpec((1, C, HT, W), lambda i: (i // GH, 0, i % GH, 0)),
            pl.BlockSpec((1, HT, W), lambda i: (i // GH, i % GH, 0)),
        ],
        out_specs=pl.BlockSpec((1, 1), lambda i: (0, 0), memory_space=pltpu.SMEM),
        out_shape=jax.ShapeDtypeStruct((1, 1), jnp.float32),
        scratch_shapes=[pltpu.VMEM((3, W), jnp.float32)],
        compiler_params=pltpu.CompilerParams(
            dimension_semantics=("arbitrary",),
        ),
    )(y_pred, y_true.astype(jnp.int32))
    return out[0, 0]
